# baseline (speedup 1.0000x reference)
import functools
import math

import jax
import jax.numpy as jnp
from jax import lax
from jax.experimental import pallas as pl
from jax.experimental.pallas import tpu as pltpu
from jax.experimental.pallas import tpu_sc as plsc

_N = 10000
_E = 320000
_D = 192
_DA = 88
_L = 4
_BR = 400
_GRID = _N // _BR
_NPAD = 10112
_DE = 16

_NC = 2
_NS = 16
_NW = _NC * _NS
_EB = 128
_NBATCH = 79
_EPW = _NBATCH * _EB
_EPAD = _EPW * _NW
_RPT = _NPAD // _NS
_DH = _D // 2
_NBATCH2 = 160
_NITER2 = _NBATCH2 // 2
_EPW2 = _NBATCH2 * _EB
_EPAD2 = _EPW2 * _NS


def _gelu(x):
    return 0.5 * x * (1.0 + lax.erf(x * (1.0 / math.sqrt(2.0))))


def _atom_body(x_ref, w_ref, b_ref, o_ref):
    o_ref[...] = (
        jnp.dot(x_ref[...], w_ref[...], preferred_element_type=jnp.float32)
        + b_ref[...]
    )


def _atom_fc(x, atom_W, atom_b):
    return pl.pallas_call(
        _atom_body,
        grid=(_GRID,),
        in_specs=[
            pl.BlockSpec((_BR, _DA), lambda i: (i, 0)),
            pl.BlockSpec((_DA, _D), lambda i: (0, 0)),
            pl.BlockSpec((1, _D), lambda i: (0, 0)),
        ],
        out_specs=pl.BlockSpec((_BR, _D), lambda i: (i, 0)),
        out_shape=jax.ShapeDtypeStruct((_N, _D), jnp.float32),
    )(x, atom_W, atom_b.reshape(1, _D))


def _layer_body(h_ref, agg_ref, ea_ref, w8_ref, w1_ref, b1_ref, w2_ref, b2_ref,
                o_ref):
    ob = (
        h_ref[...]
        + jnp.concatenate([agg_ref[0], agg_ref[1]], axis=1)
        + jnp.dot(ea_ref[0] + ea_ref[1], w8_ref[...],
                  preferred_element_type=jnp.float32)
    )
    t = jnp.dot(ob, w1_ref[...], preferred_element_type=jnp.float32) + b1_ref[...]
    t = _gelu(t)
    o_ref[...] = (
        jnp.dot(t, w2_ref[...], preferred_element_type=jnp.float32) + b2_ref[...]
    )


def _layer(h, agg2, ea2, w8, w1, b1, w2, b2):
    return pl.pallas_call(
        _layer_body,
        grid=(_GRID,),
        in_specs=[
            pl.BlockSpec((_BR, _D), lambda i: (i, 0)),
            pl.BlockSpec((2, _BR, _DH), lambda i: (0, i, 0)),
            pl.BlockSpec((2, _BR, _DE), lambda i: (0, i, 0)),
            pl.BlockSpec((_DE, _D), lambda i: (0, 0)),
            pl.BlockSpec((_D, _D), lambda i: (0, 0)),
            pl.BlockSpec((1, _D), lambda i: (0, 0)),
            pl.BlockSpec((_D, _D), lambda i: (0, 0)),
            pl.BlockSpec((1, _D), lambda i: (0, 0)),
        ],
        out_specs=pl.BlockSpec((_BR, _D), lambda i: (i, 0)),
        out_shape=jax.ShapeDtypeStruct((_N, _D), jnp.float32),
    )(h, agg2, ea2, w8, w1, b1.reshape(1, _D), w2, b2.reshape(1, _D))


def _final_body(h_ref, agg_ref, ea_ref, w8_ref, w1_ref, b1_ref, w2_ref, b2_ref,
                muW_ref, mub_ref, lvW_ref, lvb_ref, eps_ref, o_ref, pool_ref):
    i = pl.program_id(0)

    ob = (
        h_ref[...]
        + jnp.concatenate([agg_ref[0], agg_ref[1]], axis=1)
        + jnp.dot(ea_ref[0] + ea_ref[1], w8_ref[...],
                  preferred_element_type=jnp.float32)
    )
    t = jnp.dot(ob, w1_ref[...], preferred_element_type=jnp.float32) + b1_ref[...]
    t = _gelu(t)
    hn = jnp.dot(t, w2_ref[...], preferred_element_type=jnp.float32) + b2_ref[...]

    part = jnp.sum(hn, axis=0, keepdims=True)

    @pl.when(i == 0)
    def _():
        pool_ref[...] = part

    @pl.when(i > 0)
    def _():
        pool_ref[...] += part

    @pl.when(i == pl.num_programs(0) - 1)
    def _():
        pooled = pool_ref[...]
        mu = jnp.dot(pooled, muW_ref[...], preferred_element_type=jnp.float32) \
            + mub_ref[...]
        lv = jnp.dot(pooled, lvW_ref[...], preferred_element_type=jnp.float32) \
            + lvb_ref[...]
        z = mu + jnp.exp(0.5 * lv) * eps_ref[...]
        o_ref[...] = jnp.concatenate([z, mu, lv], axis=0)


def _final_layer(h, agg2, ea2, w8, w1, b1, w2, b2, mu_W, mu_b, lv_W, lv_b, eps):
    return pl.pallas_call(
        _final_body,
        grid=(_GRID,),
        in_specs=[
            pl.BlockSpec((_BR, _D), lambda i: (i, 0)),
            pl.BlockSpec((2, _BR, _DH), lambda i: (0, i, 0)),
            pl.BlockSpec((2, _BR, _DE), lambda i: (0, i, 0)),
            pl.BlockSpec((_DE, _D), lambda i: (0, 0)),
            pl.BlockSpec((_D, _D), lambda i: (0, 0)),
            pl.BlockSpec((1, _D), lambda i: (0, 0)),
            pl.BlockSpec((_D, _D), lambda i: (0, 0)),
            pl.BlockSpec((1, _D), lambda i: (0, 0)),
            pl.BlockSpec((_D, _D), lambda i: (0, 0)),
            pl.BlockSpec((1, _D), lambda i: (0, 0)),
            pl.BlockSpec((_D, _D), lambda i: (0, 0)),
            pl.BlockSpec((1, _D), lambda i: (0, 0)),
            pl.BlockSpec((1, _D), lambda i: (0, 0)),
        ],
        out_specs=pl.BlockSpec((3, _D), lambda i: (0, 0)),
        out_shape=jax.ShapeDtypeStruct((3, _D), jnp.float32),
        scratch_shapes=[pltpu.VMEM((1, _D), jnp.float32)],
    )(h, agg2, ea2, w8, w1, b1.reshape(1, _D), w2, b2.reshape(1, _D),
      mu_W, mu_b.reshape(1, _D), lv_W, lv_b.reshape(1, _D), eps.reshape(1, _D))


_SC_MESH = plsc.VectorSubcoreMesh(
    core_axis_name="c", subcore_axis_name="s", num_cores=_NC, num_subcores=_NS)

def _zero_vmem(buf, rows, cols):
    z16 = jnp.zeros((16,), jnp.float32)
    def body(r, carry):
        for c in range(cols // 16):
            buf[r, pl.ds(c * 16, 16)] = z16
        return carry
    lax.fori_loop(0, rows, body, 0)


def _zero_accum_slice(zsrc, accum, base, cols):
    for k in range((_RPT + _EB - 1) // _EB):
        r0 = k * _EB
        nr = min(_EB, _RPT - r0)
        pltpu.sync_copy(zsrc.at[pl.ds(0, nr)], accum.at[pl.ds(base + r0, nr)])


def _sc_agg_body(hv_hbm, src_hbm, dst_hbm, agg_hbm, src_v, dst_v,
                 g0, g1, accum, sem0, sem1):
    cid = lax.axis_index("c")
    sid = lax.axis_index("s")
    base = sid * _RPT

    _zero_vmem(g0, _EB, _DH)
    _zero_accum_slice(g0, accum, base, _DH)
    plsc.subcore_barrier()

    pltpu.sync_copy(src_hbm.at[cid, sid], src_v)
    pltpu.sync_copy(dst_hbm.at[sid], dst_v)

    pltpu.async_copy(hv_hbm.at[src_v.at[0]], g0, sem0)

    def batch2(i, carry):
        b0 = 2 * i
        pltpu.async_copy(hv_hbm.at[src_v.at[b0 + 1]], g1, sem1)
        pltpu.make_async_copy(hv_hbm.at[src_v.at[b0]], g0, sem0).wait()
        pltpu.sync_copy(g0, accum.at[dst_v.at[b0]], add=True)
        pltpu.async_copy(hv_hbm.at[src_v.at[b0 + 2]], g0, sem0)
        pltpu.make_async_copy(hv_hbm.at[src_v.at[b0 + 1]], g1, sem1).wait()
        pltpu.sync_copy(g1, accum.at[dst_v.at[b0 + 1]], add=True)
        return carry
    lax.fori_loop(0, _NBATCH2 // 2 - 1, batch2, 0)
    bl = _NBATCH2 - 2
    pltpu.async_copy(hv_hbm.at[src_v.at[bl + 1]], g1, sem1)
    pltpu.make_async_copy(hv_hbm.at[src_v.at[bl]], g0, sem0).wait()
    pltpu.sync_copy(g0, accum.at[dst_v.at[bl]], add=True)
    pltpu.make_async_copy(hv_hbm.at[src_v.at[bl + 1]], g1, sem1).wait()
    pltpu.sync_copy(g1, accum.at[dst_v.at[bl + 1]], add=True)
    plsc.subcore_barrier()

    pltpu.sync_copy(accum.at[pl.ds(base, _RPT)],
                    agg_hbm.at[cid, pl.ds(base, _RPT)])


def _sc_agg(hv, srcx_p, dst_p):
    return pl.kernel(
        _sc_agg_body,
        out_type=jax.ShapeDtypeStruct((2, _NPAD, _DH), jnp.float32),
        mesh=_SC_MESH,
        scratch_types=[
            pltpu.VMEM((_NBATCH2, _EB), jnp.int32),
            pltpu.VMEM((_NBATCH2, _EB), jnp.int32),
            pltpu.VMEM((_EB, _DH), jnp.float32),
            pltpu.VMEM((_EB, _DH), jnp.float32),
            pltpu.VMEM_SHARED((_NPAD, _DH), jnp.float32),
            pltpu.SemaphoreType.DMA,
            pltpu.SemaphoreType.DMA,
        ],
        compiler_params=pltpu.CompilerParams(use_tc_tiling_on_sc=False),
    )(hv, srcx_p, dst_p)


def _sc_ea_body(ea_hbm, dst_hbm, eao_hbm, dst_v, ebuf, eaccum):
    cid = lax.axis_index("c")
    sid = lax.axis_index("s")
    w = cid * _NS + sid
    base = sid * _RPT

    _zero_vmem(ebuf, _EB, _DE)
    _zero_accum_slice(ebuf, eaccum, base, _DE)
    plsc.subcore_barrier()

    pltpu.sync_copy(dst_hbm.at[w], dst_v)

    def batch(j, carry):
        pltpu.sync_copy(ea_hbm.at[w, pl.ds(j * _EB, _EB)], ebuf)
        pltpu.sync_copy(ebuf, eaccum.at[dst_v.at[j]], add=True)
        return carry
    lax.fori_loop(0, _NBATCH, batch, 0)
    plsc.subcore_barrier()

    pltpu.sync_copy(eaccum.at[pl.ds(base, _RPT)],
                    eao_hbm.at[cid, pl.ds(base, _RPT)])


def _sc_ea(ea_p, dst_p):
    return pl.kernel(
        _sc_ea_body,
        out_type=jax.ShapeDtypeStruct((2, _NPAD, _DE), jnp.float32),
        mesh=_SC_MESH,
        scratch_types=[
            pltpu.VMEM((_NBATCH, _EB), jnp.int32),
            pltpu.VMEM((_EB, _DE), jnp.float32),
            pltpu.VMEM_SHARED((_NPAD, _DE), jnp.float32),
        ],
        compiler_params=pltpu.CompilerParams(use_tc_tiling_on_sc=False),
    )(ea_p, dst_p)


def kernel(x, edge_index, edge_attr, atom_W, atom_b, edge_W, edge_b,
           mlp_W1, mlp_b1, mlp_W2, mlp_b2, mu_W, mu_b, lv_W, lv_b, eps):
    pad = _EPAD - _E
    pad2 = _EPAD2 - _E
    src = edge_index[:, 0]
    dst = edge_index[:, 1]
    src2 = jnp.concatenate([src, jnp.zeros((pad2,), jnp.int32)]) * 2
    srcx_p = jnp.stack([src2, src2 + 1]).reshape(2, _NS, _NBATCH2, _EB)
    dst_p = jnp.concatenate(
        [dst, jnp.full((pad2,), _N, jnp.int32)]).reshape(_NS, _NBATCH2, _EB)
    dst_p32 = jnp.concatenate(
        [dst, jnp.full((pad,), _N, jnp.int32)]).reshape(_NW, _NBATCH, _EB)
    ea_aug = jnp.concatenate(
        [edge_attr, jnp.ones((_E, 1), jnp.float32),
         jnp.zeros((_E, _DE - 7), jnp.float32)], axis=1)
    ea_p = jnp.concatenate(
        [ea_aug, jnp.zeros((pad, _DE), jnp.float32)]).reshape(_NW, _EPW, _DE)
    w8 = jnp.concatenate(
        [edge_W, edge_b[:, None, :], jnp.zeros((_L, _DE - 7, _D), jnp.float32)],
        axis=1)

    h = _atom_fc(x, atom_W, atom_b)
    ea2 = _sc_ea(ea_p, dst_p32)
    for l in range(_L):
        agg2 = _sc_agg(h.reshape(2 * _N, _DH), srcx_p, dst_p)
        if l < _L - 1:
            h = _layer(h, agg2, ea2, w8[l], mlp_W1[l], mlp_b1[l],
                       mlp_W2[l], mlp_b2[l])
        else:
            out = _final_layer(h, agg2, ea2, w8[l], mlp_W1[l], mlp_b1[l],
                               mlp_W2[l], mlp_b2[l],
                               mu_W, mu_b, lv_W, lv_b, eps)
    return out


# speedup vs baseline: 1.0350x; 1.0350x over previous
import functools
import math

import jax
import jax.numpy as jnp
from jax import lax
from jax.experimental import pallas as pl
from jax.experimental.pallas import tpu as pltpu
from jax.experimental.pallas import tpu_sc as plsc

_N = 10000
_E = 320000
_D = 192
_DA = 88
_L = 4
_BR = 400
_GRID = _N // _BR
_NPAD = 10112
_DE = 16

_NC = 2
_NS = 16
_NW = _NC * _NS
_EB = 128
_NBATCH = 79
_EPW = _NBATCH * _EB
_EPAD = _EPW * _NW
_RPT = _NPAD // _NS
_DH = _D // 2
_NBATCH2 = 160
_NITER2 = _NBATCH2 // 2
_EPW2 = _NBATCH2 * _EB
_EPAD2 = _EPW2 * _NS


def _gelu(x):
    return 0.5 * x * (1.0 + lax.erf(x * (1.0 / math.sqrt(2.0))))


def _atom_body(x_ref, w_ref, b_ref, o_ref):
    o_ref[...] = (
        jnp.dot(x_ref[...], w_ref[...], preferred_element_type=jnp.float32)
        + b_ref[...]
    )


def _atom_fc(x, atom_W, atom_b):
    return pl.pallas_call(
        _atom_body,
        grid=(_GRID,),
        in_specs=[
            pl.BlockSpec((_BR, _DA), lambda i: (i, 0)),
            pl.BlockSpec((_DA, _D), lambda i: (0, 0)),
            pl.BlockSpec((1, _D), lambda i: (0, 0)),
        ],
        out_specs=pl.BlockSpec((_BR, _D), lambda i: (i, 0)),
        out_shape=jax.ShapeDtypeStruct((_N, _D), jnp.float32),
    )(x, atom_W, atom_b.reshape(1, _D))


def _layer_body(h_ref, agg_ref, ea_ref, w8_ref, w1_ref, b1_ref, w2_ref, b2_ref,
                o_ref):
    ob = (
        h_ref[...]
        + jnp.concatenate([agg_ref[0], agg_ref[1]], axis=1)
        + jnp.dot(ea_ref[0] + ea_ref[1], w8_ref[...],
                  preferred_element_type=jnp.float32)
    )
    t = jnp.dot(ob, w1_ref[...], preferred_element_type=jnp.float32) + b1_ref[...]
    t = _gelu(t)
    o_ref[...] = (
        jnp.dot(t, w2_ref[...], preferred_element_type=jnp.float32) + b2_ref[...]
    )


def _layer(h, agg2, ea2, w8, w1, b1, w2, b2):
    return pl.pallas_call(
        _layer_body,
        grid=(_GRID,),
        in_specs=[
            pl.BlockSpec((_BR, _D), lambda i: (i, 0)),
            pl.BlockSpec((2, _BR, _DH), lambda i: (0, i, 0)),
            pl.BlockSpec((2, _BR, _DE), lambda i: (0, i, 0)),
            pl.BlockSpec((_DE, _D), lambda i: (0, 0)),
            pl.BlockSpec((_D, _D), lambda i: (0, 0)),
            pl.BlockSpec((1, _D), lambda i: (0, 0)),
            pl.BlockSpec((_D, _D), lambda i: (0, 0)),
            pl.BlockSpec((1, _D), lambda i: (0, 0)),
        ],
        out_specs=pl.BlockSpec((_BR, _D), lambda i: (i, 0)),
        out_shape=jax.ShapeDtypeStruct((_N, _D), jnp.float32),
    )(h, agg2, ea2, w8, w1, b1.reshape(1, _D), w2, b2.reshape(1, _D))


def _final_body(h_ref, agg_ref, ea_ref, w8_ref, w1_ref, b1_ref, w2_ref, b2_ref,
                muW_ref, mub_ref, lvW_ref, lvb_ref, eps_ref, o_ref, pool_ref):
    i = pl.program_id(0)

    ob = (
        h_ref[...]
        + jnp.concatenate([agg_ref[0], agg_ref[1]], axis=1)
        + jnp.dot(ea_ref[0] + ea_ref[1], w8_ref[...],
                  preferred_element_type=jnp.float32)
    )
    t = jnp.dot(ob, w1_ref[...], preferred_element_type=jnp.float32) + b1_ref[...]
    t = _gelu(t)
    hn = jnp.dot(t, w2_ref[...], preferred_element_type=jnp.float32) + b2_ref[...]

    part = jnp.sum(hn, axis=0, keepdims=True)

    @pl.when(i == 0)
    def _():
        pool_ref[...] = part

    @pl.when(i > 0)
    def _():
        pool_ref[...] += part

    @pl.when(i == pl.num_programs(0) - 1)
    def _():
        pooled = pool_ref[...]
        mu = jnp.dot(pooled, muW_ref[...], preferred_element_type=jnp.float32) \
            + mub_ref[...]
        lv = jnp.dot(pooled, lvW_ref[...], preferred_element_type=jnp.float32) \
            + lvb_ref[...]
        z = mu + jnp.exp(0.5 * lv) * eps_ref[...]
        o_ref[...] = jnp.concatenate([z, mu, lv], axis=0)


def _final_layer(h, agg2, ea2, w8, w1, b1, w2, b2, mu_W, mu_b, lv_W, lv_b, eps):
    return pl.pallas_call(
        _final_body,
        grid=(_GRID,),
        in_specs=[
            pl.BlockSpec((_BR, _D), lambda i: (i, 0)),
            pl.BlockSpec((2, _BR, _DH), lambda i: (0, i, 0)),
            pl.BlockSpec((2, _BR, _DE), lambda i: (0, i, 0)),
            pl.BlockSpec((_DE, _D), lambda i: (0, 0)),
            pl.BlockSpec((_D, _D), lambda i: (0, 0)),
            pl.BlockSpec((1, _D), lambda i: (0, 0)),
            pl.BlockSpec((_D, _D), lambda i: (0, 0)),
            pl.BlockSpec((1, _D), lambda i: (0, 0)),
            pl.BlockSpec((_D, _D), lambda i: (0, 0)),
            pl.BlockSpec((1, _D), lambda i: (0, 0)),
            pl.BlockSpec((_D, _D), lambda i: (0, 0)),
            pl.BlockSpec((1, _D), lambda i: (0, 0)),
            pl.BlockSpec((1, _D), lambda i: (0, 0)),
        ],
        out_specs=pl.BlockSpec((3, _D), lambda i: (0, 0)),
        out_shape=jax.ShapeDtypeStruct((3, _D), jnp.float32),
        scratch_shapes=[pltpu.VMEM((1, _D), jnp.float32)],
    )(h, agg2, ea2, w8, w1, b1.reshape(1, _D), w2, b2.reshape(1, _D),
      mu_W, mu_b.reshape(1, _D), lv_W, lv_b.reshape(1, _D), eps.reshape(1, _D))


_SC_MESH = plsc.VectorSubcoreMesh(
    core_axis_name="c", subcore_axis_name="s", num_cores=_NC, num_subcores=_NS)

def _zero_vmem(buf, rows, cols):
    z16 = jnp.zeros((16,), jnp.float32)
    def body(r, carry):
        for c in range(cols // 16):
            buf[r, pl.ds(c * 16, 16)] = z16
        return carry
    lax.fori_loop(0, rows, body, 0)


def _zero_accum_slice(zsrc, accum, base, cols):
    for k in range((_RPT + _EB - 1) // _EB):
        r0 = k * _EB
        nr = min(_EB, _RPT - r0)
        pltpu.sync_copy(zsrc.at[pl.ds(0, nr)], accum.at[pl.ds(base + r0, nr)])


def _sc_agg_body(hv_hbm, src_hbm, dst_hbm, agg_hbm, src_v, dst_v,
                 g0, g1, accum, sem0, sem1):
    cid = lax.axis_index("c")
    sid = lax.axis_index("s")
    base = sid * _RPT

    _zero_vmem(g0, _EB, _DH)
    _zero_accum_slice(g0, accum, base, _DH)
    plsc.subcore_barrier()

    pltpu.sync_copy(src_hbm.at[cid, sid], src_v)
    pltpu.sync_copy(dst_hbm.at[sid], dst_v)

    pltpu.async_copy(hv_hbm.at[src_v.at[0]], g0, sem0)

    def batch2(i, carry):
        b0 = 2 * i
        pltpu.async_copy(hv_hbm.at[src_v.at[b0 + 1]], g1, sem1)
        pltpu.make_async_copy(hv_hbm.at[src_v.at[b0]], g0, sem0).wait()
        pltpu.sync_copy(g0, accum.at[dst_v.at[b0]], add=True)
        pltpu.async_copy(hv_hbm.at[src_v.at[b0 + 2]], g0, sem0)
        pltpu.make_async_copy(hv_hbm.at[src_v.at[b0 + 1]], g1, sem1).wait()
        pltpu.sync_copy(g1, accum.at[dst_v.at[b0 + 1]], add=True)
        return carry
    lax.fori_loop(0, _NBATCH2 // 2 - 1, batch2, 0)
    bl = _NBATCH2 - 2
    pltpu.async_copy(hv_hbm.at[src_v.at[bl + 1]], g1, sem1)
    pltpu.make_async_copy(hv_hbm.at[src_v.at[bl]], g0, sem0).wait()
    pltpu.sync_copy(g0, accum.at[dst_v.at[bl]], add=True)
    pltpu.make_async_copy(hv_hbm.at[src_v.at[bl + 1]], g1, sem1).wait()
    pltpu.sync_copy(g1, accum.at[dst_v.at[bl + 1]], add=True)
    plsc.subcore_barrier()

    pltpu.sync_copy(accum.at[pl.ds(base, _RPT)],
                    agg_hbm.at[cid, pl.ds(base, _RPT)])


def _sc_agg(hv, srcx_p, dst_p):
    return pl.kernel(
        _sc_agg_body,
        out_type=jax.ShapeDtypeStruct((2, _NPAD, _DH), jnp.float32),
        mesh=_SC_MESH,
        scratch_types=[
            pltpu.VMEM((_NBATCH2, _EB), jnp.int32),
            pltpu.VMEM((_NBATCH2, _EB), jnp.int32),
            pltpu.VMEM((_EB, _DH), jnp.float32),
            pltpu.VMEM((_EB, _DH), jnp.float32),
            pltpu.VMEM_SHARED((_NPAD, _DH), jnp.float32),
            pltpu.SemaphoreType.DMA,
            pltpu.SemaphoreType.DMA,
        ],
        compiler_params=pltpu.CompilerParams(use_tc_tiling_on_sc=False),
    )(hv, srcx_p, dst_p)


def _sc_ea_body(ea_hbm, dst_hbm, eao_hbm, dst_v, ebuf, eaccum):
    cid = lax.axis_index("c")
    sid = lax.axis_index("s")
    w = cid * _NS + sid
    base = sid * _RPT

    _zero_vmem(ebuf, _EB, _DE)
    _zero_accum_slice(ebuf, eaccum, base, _DE)
    plsc.subcore_barrier()

    pltpu.sync_copy(dst_hbm.at[w], dst_v)

    def batch(j, carry):
        pltpu.sync_copy(ea_hbm.at[w, pl.ds(j * _EB, _EB)], ebuf)
        pltpu.sync_copy(ebuf, eaccum.at[dst_v.at[j]], add=True)
        return carry
    lax.fori_loop(0, _NBATCH, batch, 0)
    plsc.subcore_barrier()

    pltpu.sync_copy(eaccum.at[pl.ds(base, _RPT)],
                    eao_hbm.at[cid, pl.ds(base, _RPT)])


def _sc_ea(ea_p, dst_p):
    return pl.kernel(
        _sc_ea_body,
        out_type=jax.ShapeDtypeStruct((2, _NPAD, _DE), jnp.float32),
        mesh=_SC_MESH,
        scratch_types=[
            pltpu.VMEM((_NBATCH, _EB), jnp.int32),
            pltpu.VMEM((_EB, _DE), jnp.float32),
            pltpu.VMEM_SHARED((_NPAD, _DE), jnp.float32),
        ],
        compiler_params=pltpu.CompilerParams(use_tc_tiling_on_sc=False),
    )(ea_p, dst_p)


def kernel(x, edge_index, edge_attr, atom_W, atom_b, edge_W, edge_b,
           mlp_W1, mlp_b1, mlp_W2, mlp_b2, mu_W, mu_b, lv_W, lv_b, eps):
    pad = _EPAD - _E
    pad2 = _EPAD2 - _E
    src = edge_index[:, 0]
    dst = edge_index[:, 1]
    scratch2 = _N + (jnp.arange(pad2, dtype=jnp.int32) % (_NPAD - _N))
    scratch32 = _N + (jnp.arange(pad, dtype=jnp.int32) % (_NPAD - _N))
    src2 = jnp.concatenate([src, jnp.zeros((pad2,), jnp.int32)]) * 2
    srcx_p = jnp.stack([src2, src2 + 1]).reshape(2, _NS, _NBATCH2, _EB)
    dst_p = jnp.concatenate(
        [dst, scratch2]).reshape(_NS, _NBATCH2, _EB)
    dst_p32 = jnp.concatenate(
        [dst, scratch32]).reshape(_NW, _NBATCH, _EB)
    ea_aug = jnp.concatenate(
        [edge_attr, jnp.ones((_E, 1), jnp.float32),
         jnp.zeros((_E, _DE - 7), jnp.float32)], axis=1)
    ea_p = jnp.concatenate(
        [ea_aug, jnp.zeros((pad, _DE), jnp.float32)]).reshape(_NW, _EPW, _DE)
    w8 = jnp.concatenate(
        [edge_W, edge_b[:, None, :], jnp.zeros((_L, _DE - 7, _D), jnp.float32)],
        axis=1)

    h = _atom_fc(x, atom_W, atom_b)
    ea2 = _sc_ea(ea_p, dst_p32)
    for l in range(_L):
        agg2 = _sc_agg(h.reshape(2 * _N, _DH), srcx_p, dst_p)
        if l < _L - 1:
            h = _layer(h, agg2, ea2, w8[l], mlp_W1[l], mlp_b1[l],
                       mlp_W2[l], mlp_b2[l])
        else:
            out = _final_layer(h, agg2, ea2, w8[l], mlp_W1[l], mlp_b1[l],
                               mlp_W2[l], mlp_b2[l],
                               mu_W, mu_b, lv_W, lv_b, eps)
    return out


# speedup vs baseline: 1.8669x; 1.8038x over previous
import functools
import math

import jax
import jax.numpy as jnp
from jax import lax
from jax.experimental import pallas as pl
from jax.experimental.pallas import tpu as pltpu
from jax.experimental.pallas import tpu_sc as plsc

_N = 10000
_E = 320000
_D = 192
_DA = 88
_L = 4
_BR = 400
_GRID = _N // _BR
_NPAD = 10112
_DE = 16

_NC = 2
_NS = 16
_NW = _NC * _NS
_EB = 128
_NBATCH = 79
_EPW = _NBATCH * _EB
_EPAD = _EPW * _NW
_RPT = _NPAD // _NS
_DH = _D // 2
_NBATCH2 = 157
_EPW2 = _NBATCH2 * _EB
_EPAD2 = _EPW2 * _NS


def _gelu(x):
    return 0.5 * x * (1.0 + lax.erf(x * (1.0 / math.sqrt(2.0))))


def _atom_body(x_ref, w_ref, b_ref, o_ref):
    o_ref[...] = (
        jnp.dot(x_ref[...], w_ref[...], preferred_element_type=jnp.float32)
        + b_ref[...]
    )


def _atom_fc(x, atom_W, atom_b):
    return pl.pallas_call(
        _atom_body,
        grid=(_GRID,),
        in_specs=[
            pl.BlockSpec((_BR, _DA), lambda i: (i, 0)),
            pl.BlockSpec((_DA, _D), lambda i: (0, 0)),
            pl.BlockSpec((1, _D), lambda i: (0, 0)),
        ],
        out_specs=pl.BlockSpec((_BR, _D), lambda i: (i, 0)),
        out_shape=jax.ShapeDtypeStruct((_N, _D), jnp.float32),
    )(x, atom_W, atom_b.reshape(1, _D))


def _layer_body(h_ref, agg_ref, ea_ref, w8_ref, w1_ref, b1_ref, w2_ref, b2_ref,
                o_ref):
    ob = (
        h_ref[...]
        + jnp.concatenate([agg_ref[0], agg_ref[1]], axis=1)
        + jnp.dot(ea_ref[0] + ea_ref[1], w8_ref[...],
                  preferred_element_type=jnp.float32)
    )
    t = jnp.dot(ob, w1_ref[...], preferred_element_type=jnp.float32) + b1_ref[...]
    t = _gelu(t)
    o_ref[...] = (
        jnp.dot(t, w2_ref[...], preferred_element_type=jnp.float32) + b2_ref[...]
    )


def _layer(h, agg2, ea2, w8, w1, b1, w2, b2):
    return pl.pallas_call(
        _layer_body,
        grid=(_GRID,),
        in_specs=[
            pl.BlockSpec((_BR, _D), lambda i: (i, 0)),
            pl.BlockSpec((2, _BR, _DH), lambda i: (0, i, 0)),
            pl.BlockSpec((2, _BR, _DE), lambda i: (0, i, 0)),
            pl.BlockSpec((_DE, _D), lambda i: (0, 0)),
            pl.BlockSpec((_D, _D), lambda i: (0, 0)),
            pl.BlockSpec((1, _D), lambda i: (0, 0)),
            pl.BlockSpec((_D, _D), lambda i: (0, 0)),
            pl.BlockSpec((1, _D), lambda i: (0, 0)),
        ],
        out_specs=pl.BlockSpec((_BR, _D), lambda i: (i, 0)),
        out_shape=jax.ShapeDtypeStruct((_N, _D), jnp.float32),
    )(h, agg2, ea2, w8, w1, b1.reshape(1, _D), w2, b2.reshape(1, _D))


def _final_body(h_ref, agg_ref, ea_ref, w8_ref, w1_ref, b1_ref, w2_ref, b2_ref,
                muW_ref, mub_ref, lvW_ref, lvb_ref, eps_ref, o_ref, pool_ref):
    i = pl.program_id(0)

    ob = (
        h_ref[...]
        + jnp.concatenate([agg_ref[0], agg_ref[1]], axis=1)
        + jnp.dot(ea_ref[0] + ea_ref[1], w8_ref[...],
                  preferred_element_type=jnp.float32)
    )
    t = jnp.dot(ob, w1_ref[...], preferred_element_type=jnp.float32) + b1_ref[...]
    t = _gelu(t)
    hn = jnp.dot(t, w2_ref[...], preferred_element_type=jnp.float32) + b2_ref[...]

    part = jnp.sum(hn, axis=0, keepdims=True)

    @pl.when(i == 0)
    def _():
        pool_ref[...] = part

    @pl.when(i > 0)
    def _():
        pool_ref[...] += part

    @pl.when(i == pl.num_programs(0) - 1)
    def _():
        pooled = pool_ref[...]
        mu = jnp.dot(pooled, muW_ref[...], preferred_element_type=jnp.float32) \
            + mub_ref[...]
        lv = jnp.dot(pooled, lvW_ref[...], preferred_element_type=jnp.float32) \
            + lvb_ref[...]
        z = mu + jnp.exp(0.5 * lv) * eps_ref[...]
        o_ref[...] = jnp.concatenate([z, mu, lv], axis=0)


def _final_layer(h, agg2, ea2, w8, w1, b1, w2, b2, mu_W, mu_b, lv_W, lv_b, eps):
    return pl.pallas_call(
        _final_body,
        grid=(_GRID,),
        in_specs=[
            pl.BlockSpec((_BR, _D), lambda i: (i, 0)),
            pl.BlockSpec((2, _BR, _DH), lambda i: (0, i, 0)),
            pl.BlockSpec((2, _BR, _DE), lambda i: (0, i, 0)),
            pl.BlockSpec((_DE, _D), lambda i: (0, 0)),
            pl.BlockSpec((_D, _D), lambda i: (0, 0)),
            pl.BlockSpec((1, _D), lambda i: (0, 0)),
            pl.BlockSpec((_D, _D), lambda i: (0, 0)),
            pl.BlockSpec((1, _D), lambda i: (0, 0)),
            pl.BlockSpec((_D, _D), lambda i: (0, 0)),
            pl.BlockSpec((1, _D), lambda i: (0, 0)),
            pl.BlockSpec((_D, _D), lambda i: (0, 0)),
            pl.BlockSpec((1, _D), lambda i: (0, 0)),
            pl.BlockSpec((1, _D), lambda i: (0, 0)),
        ],
        out_specs=pl.BlockSpec((3, _D), lambda i: (0, 0)),
        out_shape=jax.ShapeDtypeStruct((3, _D), jnp.float32),
        scratch_shapes=[pltpu.VMEM((1, _D), jnp.float32)],
    )(h, agg2, ea2, w8, w1, b1.reshape(1, _D), w2, b2.reshape(1, _D),
      mu_W, mu_b.reshape(1, _D), lv_W, lv_b.reshape(1, _D), eps.reshape(1, _D))


_SC_MESH = plsc.VectorSubcoreMesh(
    core_axis_name="c", subcore_axis_name="s", num_cores=_NC, num_subcores=_NS)

def _zero_vmem(buf, rows, cols):
    z16 = jnp.zeros((16,), jnp.float32)
    def body(r, carry):
        for c in range(cols // 16):
            buf[r, pl.ds(c * 16, 16)] = z16
        return carry
    lax.fori_loop(0, rows, body, 0)


def _zero_accum_slice(zsrc, accum, base, cols):
    for k in range((_RPT + _EB - 1) // _EB):
        r0 = k * _EB
        nr = min(_EB, _RPT - r0)
        pltpu.sync_copy(zsrc.at[pl.ds(0, nr)], accum.at[pl.ds(base + r0, nr)])


def _sc_agg_body(hv_hbm, src_hbm, dst_hbm, agg_hbm, src_v, dst_v,
                 g0, g1, accum, sem0, sem1):
    cid = lax.axis_index("c")
    sid = lax.axis_index("s")
    base = sid * _RPT

    _zero_vmem(g0, _EB, _DH)
    _zero_accum_slice(g0, accum, base, _DH)
    plsc.subcore_barrier()

    pltpu.sync_copy(src_hbm.at[cid, sid], src_v)
    pltpu.sync_copy(dst_hbm.at[sid], dst_v)

    pltpu.async_copy(hv_hbm.at[src_v.at[0]], g0, sem0)

    def batch2(i, carry):
        b0 = 2 * i
        pltpu.async_copy(hv_hbm.at[src_v.at[b0 + 1]], g1, sem1)
        pltpu.make_async_copy(hv_hbm.at[src_v.at[b0]], g0, sem0).wait()
        pltpu.sync_copy(g0, accum.at[dst_v.at[b0]], add=True)
        pltpu.async_copy(hv_hbm.at[src_v.at[b0 + 2]], g0, sem0)
        pltpu.make_async_copy(hv_hbm.at[src_v.at[b0 + 1]], g1, sem1).wait()
        pltpu.sync_copy(g1, accum.at[dst_v.at[b0 + 1]], add=True)
        return carry
    lax.fori_loop(0, (_NBATCH2 - 1) // 2, batch2, 0)
    last = _NBATCH2 - 1
    pltpu.make_async_copy(hv_hbm.at[src_v.at[last]], g0, sem0).wait()
    pltpu.sync_copy(g0, accum.at[dst_v.at[last]], add=True)
    plsc.subcore_barrier()

    pltpu.sync_copy(accum.at[pl.ds(base, _RPT)],
                    agg_hbm.at[cid, pl.ds(base, _RPT)])


def _sc_agg(hv, srcx_p, dst_p):
    return pl.kernel(
        _sc_agg_body,
        out_type=jax.ShapeDtypeStruct((2, _NPAD, _DH), jnp.float32),
        mesh=_SC_MESH,
        scratch_types=[
            pltpu.VMEM((_NBATCH2, _EB), jnp.int32),
            pltpu.VMEM((_NBATCH2, _EB), jnp.int32),
            pltpu.VMEM((_EB, _DH), jnp.float32),
            pltpu.VMEM((_EB, _DH), jnp.float32),
            pltpu.VMEM_SHARED((_NPAD, _DH), jnp.float32),
            pltpu.SemaphoreType.DMA,
            pltpu.SemaphoreType.DMA,
        ],
        compiler_params=pltpu.CompilerParams(use_tc_tiling_on_sc=False),
    )(hv, srcx_p, dst_p)


def _sc_ea_body(ea_hbm, dst_hbm, eao_hbm, dst_v, ebuf, eaccum):
    cid = lax.axis_index("c")
    sid = lax.axis_index("s")
    w = cid * _NS + sid
    base = sid * _RPT

    _zero_vmem(ebuf, _EB, _DE)
    _zero_accum_slice(ebuf, eaccum, base, _DE)
    plsc.subcore_barrier()

    pltpu.sync_copy(dst_hbm.at[w], dst_v)

    def batch(j, carry):
        pltpu.sync_copy(ea_hbm.at[w, pl.ds(j * _EB, _EB)], ebuf)
        pltpu.sync_copy(ebuf, eaccum.at[dst_v.at[j]], add=True)
        return carry
    lax.fori_loop(0, _NBATCH, batch, 0)
    plsc.subcore_barrier()

    pltpu.sync_copy(eaccum.at[pl.ds(base, _RPT)],
                    eao_hbm.at[cid, pl.ds(base, _RPT)])


def _sc_ea(ea_p, dst_p):
    return pl.kernel(
        _sc_ea_body,
        out_type=jax.ShapeDtypeStruct((2, _NPAD, _DE), jnp.float32),
        mesh=_SC_MESH,
        scratch_types=[
            pltpu.VMEM((_NBATCH, _EB), jnp.int32),
            pltpu.VMEM((_EB, _DE), jnp.float32),
            pltpu.VMEM_SHARED((_NPAD, _DE), jnp.float32),
        ],
        compiler_params=pltpu.CompilerParams(use_tc_tiling_on_sc=False),
    )(ea_p, dst_p)


def kernel(x, edge_index, edge_attr, atom_W, atom_b, edge_W, edge_b,
           mlp_W1, mlp_b1, mlp_W2, mlp_b2, mu_W, mu_b, lv_W, lv_b, eps):
    pad = _EPAD - _E
    pad2 = _EPAD2 - _E
    src = edge_index[:, 0]
    dst = edge_index[:, 1]
    scratch2 = _N + (jnp.arange(pad2, dtype=jnp.int32) % (_NPAD - _N))
    scratch32 = _N + (jnp.arange(pad, dtype=jnp.int32) % (_NPAD - _N))
    src2 = jnp.concatenate([src, jnp.zeros((pad2,), jnp.int32)]) * 2
    srcx_p = jnp.stack([src2, src2 + 1]).reshape(2, _NS, _NBATCH2, _EB)
    dst_p = jnp.concatenate(
        [dst, scratch2]).reshape(_NS, _NBATCH2, _EB)
    dst_p32 = jnp.concatenate(
        [dst, scratch32]).reshape(_NW, _NBATCH, _EB)
    ea_aug = jnp.concatenate(
        [edge_attr, jnp.ones((_E, 1), jnp.float32),
         jnp.zeros((_E, _DE - 7), jnp.float32)], axis=1)
    ea_p = jnp.concatenate(
        [ea_aug, jnp.zeros((pad, _DE), jnp.float32)]).reshape(_NW, _EPW, _DE)
    w8 = jnp.concatenate(
        [edge_W, edge_b[:, None, :], jnp.zeros((_L, _DE - 7, _D), jnp.float32)],
        axis=1)

    h = _atom_fc(x, atom_W, atom_b)
    ea2 = _sc_ea(ea_p, dst_p32)
    for l in range(_L):
        agg2 = _sc_agg(h.reshape(2 * _N, _DH), srcx_p, dst_p)
        if l < _L - 1:
            h = _layer(h, agg2, ea2, w8[l], mlp_W1[l], mlp_b1[l],
                       mlp_W2[l], mlp_b2[l])
        else:
            out = _final_layer(h, agg2, ea2, w8[l], mlp_W1[l], mlp_b1[l],
                               mlp_W2[l], mlp_b2[l],
                               mu_W, mu_b, lv_W, lv_b, eps)
    return out


# speedup vs baseline: 1.9302x; 1.0339x over previous
import functools
import math

import jax
import jax.numpy as jnp
from jax import lax
from jax.experimental import pallas as pl
from jax.experimental.pallas import tpu as pltpu
from jax.experimental.pallas import tpu_sc as plsc

_N = 10000
_E = 320000
_D = 192
_DA = 88
_L = 4
_BR = 400
_GRID = _N // _BR
_NPAD = 10112
_DE = 16

_NC = 2
_NS = 16
_NW = _NC * _NS
_EB = 128
_NBATCH = 79
_EPW = _NBATCH * _EB
_EPAD = _EPW * _NW
_RPT = _NPAD // _NS
_DH = _D // 2
_NBATCH2 = 157
_EPW2 = _NBATCH2 * _EB
_EPAD2 = _EPW2 * _NS


def _gelu(x):
    return 0.5 * x * (1.0 + lax.erf(x * (1.0 / math.sqrt(2.0))))


def _atom_body(x_ref, w_ref, b_ref, o_ref):
    h = (
        jnp.dot(x_ref[...], w_ref[...], preferred_element_type=jnp.float32)
        + b_ref[...]
    )
    o_ref[0] = h[:, :_DH]
    o_ref[1] = h[:, _DH:]


def _atom_fc(x, atom_W, atom_b):
    return pl.pallas_call(
        _atom_body,
        grid=(_GRID,),
        in_specs=[
            pl.BlockSpec((_BR, _DA), lambda i: (i, 0)),
            pl.BlockSpec((_DA, _D), lambda i: (0, 0)),
            pl.BlockSpec((1, _D), lambda i: (0, 0)),
        ],
        out_specs=pl.BlockSpec((2, _BR, _DH), lambda i: (0, i, 0)),
        out_shape=jax.ShapeDtypeStruct((2, _N, _DH), jnp.float32),
    )(x, atom_W, atom_b.reshape(1, _D))


def _layer_body(h_ref, agg_ref, ea_ref, w8_ref, w1_ref, b1_ref, w2_ref, b2_ref,
                o_ref):
    ob = (
        jnp.concatenate([h_ref[0], h_ref[1]], axis=1)
        + jnp.concatenate([agg_ref[0], agg_ref[1]], axis=1)
        + jnp.dot(ea_ref[0] + ea_ref[1], w8_ref[...],
                  preferred_element_type=jnp.float32)
    )
    t = jnp.dot(ob, w1_ref[...], preferred_element_type=jnp.float32) + b1_ref[...]
    t = _gelu(t)
    hn = (
        jnp.dot(t, w2_ref[...], preferred_element_type=jnp.float32) + b2_ref[...]
    )
    o_ref[0] = hn[:, :_DH]
    o_ref[1] = hn[:, _DH:]


def _layer(h, agg2, ea2, w8, w1, b1, w2, b2):
    return pl.pallas_call(
        _layer_body,
        grid=(_GRID,),
        in_specs=[
            pl.BlockSpec((2, _BR, _DH), lambda i: (0, i, 0)),
            pl.BlockSpec((2, _BR, _DH), lambda i: (0, i, 0)),
            pl.BlockSpec((2, _BR, _DE), lambda i: (0, i, 0)),
            pl.BlockSpec((_DE, _D), lambda i: (0, 0)),
            pl.BlockSpec((_D, _D), lambda i: (0, 0)),
            pl.BlockSpec((1, _D), lambda i: (0, 0)),
            pl.BlockSpec((_D, _D), lambda i: (0, 0)),
            pl.BlockSpec((1, _D), lambda i: (0, 0)),
        ],
        out_specs=pl.BlockSpec((2, _BR, _DH), lambda i: (0, i, 0)),
        out_shape=jax.ShapeDtypeStruct((2, _N, _DH), jnp.float32),
    )(h, agg2, ea2, w8, w1, b1.reshape(1, _D), w2, b2.reshape(1, _D))


def _final_body(h_ref, agg_ref, ea_ref, w8_ref, w1_ref, b1_ref, w2_ref, b2_ref,
                muW_ref, mub_ref, lvW_ref, lvb_ref, eps_ref, o_ref, pool_ref):
    i = pl.program_id(0)

    ob = (
        jnp.concatenate([h_ref[0], h_ref[1]], axis=1)
        + jnp.concatenate([agg_ref[0], agg_ref[1]], axis=1)
        + jnp.dot(ea_ref[0] + ea_ref[1], w8_ref[...],
                  preferred_element_type=jnp.float32)
    )
    t = jnp.dot(ob, w1_ref[...], preferred_element_type=jnp.float32) + b1_ref[...]
    t = _gelu(t)
    hn = jnp.dot(t, w2_ref[...], preferred_element_type=jnp.float32) + b2_ref[...]

    part = jnp.sum(hn, axis=0, keepdims=True)

    @pl.when(i == 0)
    def _():
        pool_ref[...] = part

    @pl.when(i > 0)
    def _():
        pool_ref[...] += part

    @pl.when(i == pl.num_programs(0) - 1)
    def _():
        pooled = pool_ref[...]
        mu = jnp.dot(pooled, muW_ref[...], preferred_element_type=jnp.float32) \
            + mub_ref[...]
        lv = jnp.dot(pooled, lvW_ref[...], preferred_element_type=jnp.float32) \
            + lvb_ref[...]
        z = mu + jnp.exp(0.5 * lv) * eps_ref[...]
        o_ref[...] = jnp.concatenate([z, mu, lv], axis=0)


def _final_layer(h, agg2, ea2, w8, w1, b1, w2, b2, mu_W, mu_b, lv_W, lv_b, eps):
    return pl.pallas_call(
        _final_body,
        grid=(_GRID,),
        in_specs=[
            pl.BlockSpec((2, _BR, _DH), lambda i: (0, i, 0)),
            pl.BlockSpec((2, _BR, _DH), lambda i: (0, i, 0)),
            pl.BlockSpec((2, _BR, _DE), lambda i: (0, i, 0)),
            pl.BlockSpec((_DE, _D), lambda i: (0, 0)),
            pl.BlockSpec((_D, _D), lambda i: (0, 0)),
            pl.BlockSpec((1, _D), lambda i: (0, 0)),
            pl.BlockSpec((_D, _D), lambda i: (0, 0)),
            pl.BlockSpec((1, _D), lambda i: (0, 0)),
            pl.BlockSpec((_D, _D), lambda i: (0, 0)),
            pl.BlockSpec((1, _D), lambda i: (0, 0)),
            pl.BlockSpec((_D, _D), lambda i: (0, 0)),
            pl.BlockSpec((1, _D), lambda i: (0, 0)),
            pl.BlockSpec((1, _D), lambda i: (0, 0)),
        ],
        out_specs=pl.BlockSpec((3, _D), lambda i: (0, 0)),
        out_shape=jax.ShapeDtypeStruct((3, _D), jnp.float32),
        scratch_shapes=[pltpu.VMEM((1, _D), jnp.float32)],
    )(h, agg2, ea2, w8, w1, b1.reshape(1, _D), w2, b2.reshape(1, _D),
      mu_W, mu_b.reshape(1, _D), lv_W, lv_b.reshape(1, _D), eps.reshape(1, _D))


_SC_MESH = plsc.VectorSubcoreMesh(
    core_axis_name="c", subcore_axis_name="s", num_cores=_NC, num_subcores=_NS)

def _zero_vmem(buf, rows, cols):
    z16 = jnp.zeros((16,), jnp.float32)
    def body(r, carry):
        for c in range(cols // 16):
            buf[r, pl.ds(c * 16, 16)] = z16
        return carry
    lax.fori_loop(0, rows, body, 0)


def _zero_accum_slice(zsrc, accum, base, cols):
    for k in range((_RPT + _EB - 1) // _EB):
        r0 = k * _EB
        nr = min(_EB, _RPT - r0)
        pltpu.sync_copy(zsrc.at[pl.ds(0, nr)], accum.at[pl.ds(base + r0, nr)])


def _sc_agg_body(hv_hbm, src_hbm, dst_hbm, agg_hbm, src_v, dst_v,
                 g0, g1, accum, sem0, sem1):
    cid = lax.axis_index("c")
    sid = lax.axis_index("s")
    base = sid * _RPT

    _zero_vmem(g0, _EB, _DH)
    _zero_accum_slice(g0, accum, base, _DH)
    plsc.subcore_barrier()

    pltpu.sync_copy(src_hbm.at[cid, sid], src_v)
    pltpu.sync_copy(dst_hbm.at[sid], dst_v)

    pltpu.async_copy(hv_hbm.at[src_v.at[0]], g0, sem0)

    def batch2(i, carry):
        b0 = 2 * i
        pltpu.async_copy(hv_hbm.at[src_v.at[b0 + 1]], g1, sem1)
        pltpu.make_async_copy(hv_hbm.at[src_v.at[b0]], g0, sem0).wait()
        pltpu.sync_copy(g0, accum.at[dst_v.at[b0]], add=True)
        pltpu.async_copy(hv_hbm.at[src_v.at[b0 + 2]], g0, sem0)
        pltpu.make_async_copy(hv_hbm.at[src_v.at[b0 + 1]], g1, sem1).wait()
        pltpu.sync_copy(g1, accum.at[dst_v.at[b0 + 1]], add=True)
        return carry
    lax.fori_loop(0, (_NBATCH2 - 1) // 2, batch2, 0)
    last = _NBATCH2 - 1
    pltpu.make_async_copy(hv_hbm.at[src_v.at[last]], g0, sem0).wait()
    pltpu.sync_copy(g0, accum.at[dst_v.at[last]], add=True)
    plsc.subcore_barrier()

    pltpu.sync_copy(accum.at[pl.ds(base, _RPT)],
                    agg_hbm.at[cid, pl.ds(base, _RPT)])


def _sc_agg(hv, srcx_p, dst_p):
    return pl.kernel(
        _sc_agg_body,
        out_type=jax.ShapeDtypeStruct((2, _NPAD, _DH), jnp.float32),
        mesh=_SC_MESH,
        scratch_types=[
            pltpu.VMEM((_NBATCH2, _EB), jnp.int32),
            pltpu.VMEM((_NBATCH2, _EB), jnp.int32),
            pltpu.VMEM((_EB, _DH), jnp.float32),
            pltpu.VMEM((_EB, _DH), jnp.float32),
            pltpu.VMEM_SHARED((_NPAD, _DH), jnp.float32),
            pltpu.SemaphoreType.DMA,
            pltpu.SemaphoreType.DMA,
        ],
        compiler_params=pltpu.CompilerParams(use_tc_tiling_on_sc=False),
    )(hv, srcx_p, dst_p)


def _sc_ea_body(ea_hbm, dst_hbm, eao_hbm, dst_v, ebuf, eaccum):
    cid = lax.axis_index("c")
    sid = lax.axis_index("s")
    w = cid * _NS + sid
    base = sid * _RPT

    _zero_vmem(ebuf, _EB, _DE)
    _zero_accum_slice(ebuf, eaccum, base, _DE)
    plsc.subcore_barrier()

    pltpu.sync_copy(dst_hbm.at[w], dst_v)

    def batch(j, carry):
        pltpu.sync_copy(ea_hbm.at[w, pl.ds(j * _EB, _EB)], ebuf)
        pltpu.sync_copy(ebuf, eaccum.at[dst_v.at[j]], add=True)
        return carry
    lax.fori_loop(0, _NBATCH, batch, 0)
    plsc.subcore_barrier()

    pltpu.sync_copy(eaccum.at[pl.ds(base, _RPT)],
                    eao_hbm.at[cid, pl.ds(base, _RPT)])


def _sc_ea(ea_p, dst_p):
    return pl.kernel(
        _sc_ea_body,
        out_type=jax.ShapeDtypeStruct((2, _NPAD, _DE), jnp.float32),
        mesh=_SC_MESH,
        scratch_types=[
            pltpu.VMEM((_NBATCH, _EB), jnp.int32),
            pltpu.VMEM((_EB, _DE), jnp.float32),
            pltpu.VMEM_SHARED((_NPAD, _DE), jnp.float32),
        ],
        compiler_params=pltpu.CompilerParams(use_tc_tiling_on_sc=False),
    )(ea_p, dst_p)


def kernel(x, edge_index, edge_attr, atom_W, atom_b, edge_W, edge_b,
           mlp_W1, mlp_b1, mlp_W2, mlp_b2, mu_W, mu_b, lv_W, lv_b, eps):
    pad = _EPAD - _E
    pad2 = _EPAD2 - _E
    src = edge_index[:, 0]
    dst = edge_index[:, 1]
    scratch2 = _N + (jnp.arange(pad2, dtype=jnp.int32) % (_NPAD - _N))
    scratch32 = _N + (jnp.arange(pad, dtype=jnp.int32) % (_NPAD - _N))
    src2 = jnp.concatenate([src, jnp.zeros((pad2,), jnp.int32)])
    srcx_p = jnp.stack([src2, src2 + _N]).reshape(2, _NS, _NBATCH2, _EB)
    dst_p = jnp.concatenate(
        [dst, scratch2]).reshape(_NS, _NBATCH2, _EB)
    dst_p32 = jnp.concatenate(
        [dst, scratch32]).reshape(_NW, _NBATCH, _EB)
    ea_aug = jnp.concatenate(
        [edge_attr, jnp.ones((_E, 1), jnp.float32),
         jnp.zeros((_E, _DE - 7), jnp.float32)], axis=1)
    ea_p = jnp.concatenate(
        [ea_aug, jnp.zeros((pad, _DE), jnp.float32)]).reshape(_NW, _EPW, _DE)
    w8 = jnp.concatenate(
        [edge_W, edge_b[:, None, :], jnp.zeros((_L, _DE - 7, _D), jnp.float32)],
        axis=1)

    h = _atom_fc(x, atom_W, atom_b)
    ea2 = _sc_ea(ea_p, dst_p32)
    for l in range(_L):
        agg2 = _sc_agg(h.reshape(2 * _N, _DH), srcx_p, dst_p)
        if l < _L - 1:
            h = _layer(h, agg2, ea2, w8[l], mlp_W1[l], mlp_b1[l],
                       mlp_W2[l], mlp_b2[l])
        else:
            out = _final_layer(h, agg2, ea2, w8[l], mlp_W1[l], mlp_b1[l],
                               mlp_W2[l], mlp_b2[l],
                               mu_W, mu_b, lv_W, lv_b, eps)
    return out
